# baseline (speedup 1.0000x reference)
import functools

import jax
import jax.numpy as jnp
from jax import lax
from jax.experimental import pallas as pl
from jax.experimental.pallas import tpu as pltpu
from jax.experimental.pallas import tpu_sc as plsc

NC = 2
NS = 16
NW = NC * NS
L = 16

R = 16
CH = R * 1024


def _make_sc_add(B, T, E):
    TW = T // NW
    n_chunks = TW // R
    mesh = plsc.VectorSubcoreMesh(core_axis_name="c", subcore_axis_name="s")

    @functools.partial(
        pl.kernel,
        out_type=jax.ShapeDtypeStruct((B * T * E,), jnp.float32),
        mesh=mesh,
        scratch_types=[
            pltpu.VMEM((CH,), jnp.float32),
            pltpu.VMEM((CH,), jnp.float32),
        ],
    )
    def k(x_hbm, emb_hbm, out_hbm, embbuf, xbuf):
        c = lax.axis_index("c")
        s = lax.axis_index("s")
        wid = s * NC + c
        base = wid * TW

        def chunk_body(j, _):
            t0 = (base + j * R) * E
            pltpu.sync_copy(emb_hbm.at[pl.ds(t0, CH)], embbuf)

            def batch_body(b, _):
                off = b * (T * E) + t0
                pltpu.sync_copy(x_hbm.at[pl.ds(off, CH)], xbuf)

                def add_body(i, _):
                    sl = pl.ds(i * L, L)
                    plsc.addupdate(xbuf.at[sl], embbuf[sl])
                    return 0

                lax.fori_loop(0, CH // L, add_body, 0)
                pltpu.sync_copy(xbuf, out_hbm.at[pl.ds(off, CH)])
                return 0

            lax.fori_loop(0, B, batch_body, 0)
            return 0

        lax.fori_loop(0, n_chunks, chunk_body, 0)

    return k


def kernel(x, emb):
    B, T, E = x.shape
    sc_add = _make_sc_add(B, T, E)
    out = sc_add(x.reshape(-1), emb[:T].reshape(-1))
    return out.reshape(B, T, E)


# speedup vs baseline: 1.7371x; 1.7371x over previous
import functools

import jax
import jax.numpy as jnp
from jax import lax
from jax.experimental import pallas as pl
from jax.experimental.pallas import tpu as pltpu
from jax.experimental.pallas import tpu_sc as plsc

NC = 2
NS = 16
NW = NC * NS
L = 16

R = 16
E = 1024
CH = R * E
UN = 8


def _add_chunk(embbuf, xbuf):
    def body(i, _):
        base = i * (L * UN)
        for k in range(UN):
            sl = pl.ds(base + k * L, L)
            plsc.addupdate(xbuf.at[sl], embbuf[sl])
        return 0

    lax.fori_loop(0, CH // (L * UN), body, 0)


def _make_sc_add(B, T):
    TW = T // NW
    n_chunks = TW // R
    n_tasks = n_chunks * B
    mesh = plsc.VectorSubcoreMesh(core_axis_name="c", subcore_axis_name="s")

    @functools.partial(
        pl.kernel,
        out_type=jax.ShapeDtypeStruct((B * T * E,), jnp.float32),
        mesh=mesh,
        scratch_types=[
            pltpu.VMEM((CH,), jnp.float32),
            pltpu.VMEM((CH,), jnp.float32),
            pltpu.VMEM((CH,), jnp.float32),
            pltpu.VMEM((CH,), jnp.float32),
            pltpu.VMEM((CH,), jnp.float32),
            pltpu.SemaphoreType.DMA,
            pltpu.SemaphoreType.DMA,
            pltpu.SemaphoreType.DMA,
            pltpu.SemaphoreType.DMA,
            pltpu.SemaphoreType.DMA,
            pltpu.SemaphoreType.DMA,
            pltpu.SemaphoreType.DMA,
            pltpu.SemaphoreType.DMA,
        ],
    )
    def k(x_hbm, emb_hbm, out_hbm, eb0, eb1, xb0, xb1, xb2,
          es0, es1, ls0, ls1, ls2, ss0, ss1, ss2):
        ebufs, esems = (eb0, eb1), (es0, es1)
        xbufs, lsems, ssems = (xb0, xb1, xb2), (ls0, ls1, ls2), (ss0, ss1, ss2)

        c = lax.axis_index("c")
        s = lax.axis_index("s")
        wid = s * NC + c
        base = wid * TW

        def emb_off(j):
            return (base + j * R) * E

        def x_off(n):
            j, b = n // B, n % B
            return b * (T * E) + emb_off(j)

        pltpu.async_copy(emb_hbm.at[pl.ds(emb_off(0), CH)], ebufs[0], esems[0])
        pltpu.async_copy(x_hbm.at[pl.ds(x_off(0), CH)], xbufs[0], lsems[0])

        for n in range(n_tasks):
            j, b = n // B, n % B
            p = n % 3
            if b == 0:
                e = j % 2
                pltpu.make_async_copy(
                    emb_hbm.at[pl.ds(emb_off(j), CH)], ebufs[e], esems[e]
                ).wait()
                if j + 1 < n_chunks:
                    e2 = (j + 1) % 2
                    pltpu.async_copy(
                        emb_hbm.at[pl.ds(emb_off(j + 1), CH)], ebufs[e2], esems[e2]
                    )
            pltpu.make_async_copy(
                x_hbm.at[pl.ds(x_off(n), CH)], xbufs[p], lsems[p]
            ).wait()
            if n + 1 < n_tasks:
                q = (n + 1) % 3
                if n >= 2:
                    pltpu.make_async_copy(
                        xbufs[q], out_hbm.at[pl.ds(x_off(n - 2), CH)], ssems[q]
                    ).wait()
                pltpu.async_copy(
                    x_hbm.at[pl.ds(x_off(n + 1), CH)], xbufs[q], lsems[q]
                )
            _add_chunk(ebufs[j % 2], xbufs[p])
            pltpu.async_copy(xbufs[p], out_hbm.at[pl.ds(x_off(n), CH)], ssems[p])

        for m in range(max(0, n_tasks - 3), n_tasks):
            pm = m % 3
            pltpu.make_async_copy(
                xbufs[pm], out_hbm.at[pl.ds(x_off(m), CH)], ssems[pm]
            ).wait()

    return k


def kernel(x, emb):
    B, T, Ex = x.shape
    assert Ex == E
    sc_add = _make_sc_add(B, T)
    out = sc_add(x.reshape(-1), emb[:T].reshape(-1))
    return out.reshape(B, T, Ex)


# speedup vs baseline: 1.7400x; 1.0016x over previous
import functools

import jax
import jax.numpy as jnp
from jax import lax
from jax.experimental import pallas as pl
from jax.experimental.pallas import tpu as pltpu
from jax.experimental.pallas import tpu_sc as plsc

NC = 2
NS = 16
NW = NC * NS
L = 16

R = 16
E = 1024
CH = R * E
UN = 8


def _add_chunk(embbuf, xbuf):
    @plsc.parallel_loop(0, CH, step=L, unroll=UN)
    def body(i):
        sl = pl.ds(i, L)
        plsc.addupdate(xbuf.at[sl], embbuf[sl])


def _make_sc_add(B, T):
    TW = T // NW
    n_chunks = TW // R
    n_tasks = n_chunks * B
    mesh = plsc.VectorSubcoreMesh(core_axis_name="c", subcore_axis_name="s")

    @functools.partial(
        pl.kernel,
        out_type=jax.ShapeDtypeStruct((B * T * E,), jnp.float32),
        mesh=mesh,
        scratch_types=[
            pltpu.VMEM((CH,), jnp.float32),
            pltpu.VMEM((CH,), jnp.float32),
            pltpu.VMEM((CH,), jnp.float32),
            pltpu.VMEM((CH,), jnp.float32),
            pltpu.VMEM((CH,), jnp.float32),
            pltpu.SemaphoreType.DMA,
            pltpu.SemaphoreType.DMA,
            pltpu.SemaphoreType.DMA,
            pltpu.SemaphoreType.DMA,
            pltpu.SemaphoreType.DMA,
            pltpu.SemaphoreType.DMA,
            pltpu.SemaphoreType.DMA,
            pltpu.SemaphoreType.DMA,
        ],
    )
    def k(x_hbm, emb_hbm, out_hbm, eb0, eb1, xb0, xb1, xb2,
          es0, es1, ls0, ls1, ls2, ss0, ss1, ss2):
        ebufs, esems = (eb0, eb1), (es0, es1)
        xbufs, lsems, ssems = (xb0, xb1, xb2), (ls0, ls1, ls2), (ss0, ss1, ss2)

        c = lax.axis_index("c")
        s = lax.axis_index("s")
        wid = s * NC + c
        base = wid * TW

        def emb_off(j):
            return (base + j * R) * E

        def x_off(n):
            j, b = n // B, n % B
            return b * (T * E) + emb_off(j)

        pltpu.async_copy(emb_hbm.at[pl.ds(emb_off(0), CH)], ebufs[0], esems[0])
        pltpu.async_copy(x_hbm.at[pl.ds(x_off(0), CH)], xbufs[0], lsems[0])

        for n in range(n_tasks):
            j, b = n // B, n % B
            p = n % 3
            if b == 0:
                e = j % 2
                pltpu.make_async_copy(
                    emb_hbm.at[pl.ds(emb_off(j), CH)], ebufs[e], esems[e]
                ).wait()
                if j + 1 < n_chunks:
                    e2 = (j + 1) % 2
                    pltpu.async_copy(
                        emb_hbm.at[pl.ds(emb_off(j + 1), CH)], ebufs[e2], esems[e2]
                    )
            pltpu.make_async_copy(
                x_hbm.at[pl.ds(x_off(n), CH)], xbufs[p], lsems[p]
            ).wait()
            if n + 1 < n_tasks:
                q = (n + 1) % 3
                if n >= 2:
                    pltpu.make_async_copy(
                        xbufs[q], out_hbm.at[pl.ds(x_off(n - 2), CH)], ssems[q]
                    ).wait()
                pltpu.async_copy(
                    x_hbm.at[pl.ds(x_off(n + 1), CH)], xbufs[q], lsems[q]
                )
            _add_chunk(ebufs[j % 2], xbufs[p])
            pltpu.async_copy(xbufs[p], out_hbm.at[pl.ds(x_off(n), CH)], ssems[p])

        for m in range(max(0, n_tasks - 3), n_tasks):
            pm = m % 3
            pltpu.make_async_copy(
                xbufs[pm], out_hbm.at[pl.ds(x_off(m), CH)], ssems[pm]
            ).wait()

    return k


def kernel(x, emb):
    B, T, Ex = x.shape
    assert Ex == E
    sc_add = _make_sc_add(B, T)
    out = sc_add(x.reshape(-1), emb[:T].reshape(-1))
    return out.reshape(B, T, Ex)


# speedup vs baseline: 7.5145x; 4.3187x over previous
import jax
import jax.numpy as jnp
from jax.experimental import pallas as pl

BT = 2048


def _add_block(x_ref, e_ref, o_ref):
    o_ref[...] = x_ref[...] + e_ref[...]


def kernel(x, emb):
    B, T, E = x.shape
    return pl.pallas_call(
        _add_block,
        grid=(T // BT, B),
        in_specs=[
            pl.BlockSpec((1, BT, E), lambda i, j: (j, i, 0)),
            pl.BlockSpec((BT, E), lambda i, j: (i, 0)),
        ],
        out_specs=pl.BlockSpec((1, BT, E), lambda i, j: (j, i, 0)),
        out_shape=jax.ShapeDtypeStruct((B, T, E), x.dtype),
    )(x, emb)
